# Initial kernel scaffold; baseline (speedup 1.0000x reference)
import jax
import jax.numpy as jnp
from jax.experimental import pallas as pl


def kernel(meta_indices, table):
    raise NotImplementedError("write your pallas kernel here")


# speedup vs baseline: 1.6187x; 1.6187x over previous
import functools

import jax
import jax.numpy as jnp
from jax import lax
from jax.experimental import pallas as pl
from jax.experimental.pallas import tpu as pltpu
from jax.experimental.pallas import tpu_sc as plsc

NC, NS, L = 2, 16, 16
NW = NC * NS
B, S, M, D = 4096, 50, 5, 64
G = B * S
N = G * M
ROWS_PER_DMA = 128
DMAS_PER_CHUNK = 5
CHUNK = ROWS_PER_DMA * DMAS_PER_CHUNK
GROUPS_PER_CHUNK = CHUNK // M
NIDX_W = N // NW
NCHUNK = NIDX_W // CHUNK
NBUF = 2
IDX_ROWS_W = NIDX_W // ROWS_PER_DMA
GROUPS_W = NIDX_W // M


def _body(idx_hbm, table_hbm, out_hbm, idx_v, rows_v, out_v, sem0, sem1):
  sems = (sem0, sem1)
  wid = lax.axis_index("s") * NC + lax.axis_index("c")

  def fire(b, c):
    i0 = wid * NIDX_W + c * CHUNK
    pltpu.sync_copy(idx_hbm.at[pl.ds(i0, CHUNK)], idx_v.at[b])
    for j in range(DMAS_PER_CHUNK):
      pltpu.async_copy(
          table_hbm.at[idx_v.at[b, pl.ds(j * ROWS_PER_DMA, ROWS_PER_DMA)]],
          rows_v.at[b, pl.ds(j * ROWS_PER_DMA, ROWS_PER_DMA)],
          sems[b],
      )

  def drain(b):
    pltpu.make_async_copy(
        out_hbm.at[pl.ds(0, CHUNK)], rows_v.at[b], sems[b]
    ).wait()

  def reduce_store(b, c):
    def grp(g, carry):
      r = g * M
      for d in range(D // L):
        sl = pl.ds(d * L, L)
        acc = rows_v[b, r, sl]
        for m in range(1, M):
          acc = acc + rows_v[b, r + m, sl]
        out_v[g, sl] = acc * (1.0 / M)
      return carry

    lax.fori_loop(0, GROUPS_PER_CHUNK, grp, 0, unroll=2)
    o0 = wid * GROUPS_W + c * GROUPS_PER_CHUNK
    pltpu.sync_copy(out_v, out_hbm.at[pl.ds(o0, GROUPS_PER_CHUNK)])

  for b in range(NBUF):
    fire(b, b)

  def step(s, carry):
    for b in range(NBUF):
      c = s * NBUF + b
      drain(b)
      reduce_store(b, c)
      cn = c + NBUF

      @pl.when(cn < NCHUNK)
      def _():
        fire(b, cn)

    return carry

  lax.fori_loop(0, NCHUNK // NBUF, step, 0)


_sc_call = pl.kernel(
    _body,
    out_type=jax.ShapeDtypeStruct((G, D), jnp.float32),
    mesh=plsc.VectorSubcoreMesh(
        core_axis_name="c", subcore_axis_name="s", num_cores=NC,
        num_subcores=NS),
    scratch_types=[
        pltpu.VMEM((NBUF, CHUNK), jnp.int32),
        pltpu.VMEM((NBUF, CHUNK, D), jnp.float32),
        pltpu.VMEM((GROUPS_PER_CHUNK, D), jnp.float32),
        pltpu.SemaphoreType.DMA,
        pltpu.SemaphoreType.DMA,
    ],
    compiler_params=pltpu.CompilerParams(use_tc_tiling_on_sc=False),
)


@jax.jit
def _run(idx2, table):
  return _sc_call(idx2, table)


def kernel(meta_indices, table):
  idx2 = meta_indices.astype(jnp.int32).reshape(N)
  out = _run(idx2, table)
  return out.reshape(B, S, D)


# speedup vs baseline: 1.6310x; 1.0076x over previous
import jax
import jax.numpy as jnp
from jax import lax
from jax.experimental import pallas as pl
from jax.experimental.pallas import tpu as pltpu
from jax.experimental.pallas import tpu_sc as plsc

NC, NS, L = 2, 16, 16
NW = NC * NS
B, S, M, D = 4096, 50, 5, 64
BATCHES_W = B // NW
CB = 2
CHUNK = CB * S * M
NCHUNK = BATCHES_W // CB
PAIR = 2 * CHUNK
NBUF = 2
PIECE = 128
EVEN_SPLITS = ((0, 0), (128, 128), (256, 256), (376, 376))
ODD_SPLITS = ((496, 0), (624, 128), (744, 248), (872, 376))
NROWS = 512


def _body(idx_hbm, table_hbm, out3_hbm, idx_v, rows_v, out_v, sem0, sem1):
  sems = (sem0, sem1)
  wid = lax.axis_index("s") * NC + lax.axis_index("c")
  wb0 = wid * BATCHES_W
  wi0 = wid * BATCHES_W * S * M

  def fire(b, c, ps):
    if b == 0:
      pltpu.sync_copy(
          idx_hbm.at[pl.ds(wi0 + c * CHUNK, PAIR)],
          idx_v.at[ps],
      )
    splits = EVEN_SPLITS if b == 0 else ODD_SPLITS
    for soff, doff in splits:
      pltpu.async_copy(
          table_hbm.at[idx_v.at[ps, pl.ds(soff, PIECE)]],
          rows_v.at[b, pl.ds(doff, PIECE)],
          sems[b],
      )

  def drain(b):
    pltpu.make_async_copy(
        table_hbm.at[pl.ds(0, NROWS)], rows_v.at[b], sems[b]
    ).wait()

  def reduce_store(b, c):
    ro = 0 if b == 0 else 4

    for gb in range(CB):
      def grp(ss, carry, gb=gb):
        r = ro + (gb * S + ss) * M
        for d in range(D // L):
          sl = pl.ds(d * L, L)
          acc = rows_v[b, r, sl]
          for m in range(1, M):
            acc = acc + rows_v[b, r + m, sl]
          out_v[gb, ss, sl] = acc * (1.0 / M)
        return carry

      lax.fori_loop(0, S, grp, 0, unroll=2)
    pltpu.sync_copy(out_v, out3_hbm.at[pl.ds(wb0 + c * CB, CB)])

  fire(0, 0, 0)
  fire(1, 1, 0)

  def step(s, carry):
    pn = (s + 1) & 1
    for b in range(NBUF):
      c = s * NBUF + b
      drain(b)
      reduce_store(b, c)
      cn = c + NBUF

      @pl.when(cn < NCHUNK)
      def _():
        fire(b, cn, pn)

    return carry

  lax.fori_loop(0, NCHUNK // NBUF, step, 0)


_sc_call = pl.kernel(
    _body,
    out_type=jax.ShapeDtypeStruct((B, S, D), jnp.float32),
    mesh=plsc.VectorSubcoreMesh(
        core_axis_name="c", subcore_axis_name="s", num_cores=NC,
        num_subcores=NS),
    scratch_types=[
        pltpu.VMEM((2, PAIR), jnp.int32),
        pltpu.VMEM((NBUF, NROWS, D), jnp.float32),
        pltpu.VMEM((CB, S, D), jnp.float32),
        pltpu.SemaphoreType.DMA,
        pltpu.SemaphoreType.DMA,
    ],
    compiler_params=pltpu.CompilerParams(use_tc_tiling_on_sc=False),
)


@jax.jit
def _run(idx_flat, table):
  return _sc_call(idx_flat, table)


def kernel(meta_indices, table):
  idx_flat = meta_indices.astype(jnp.int32).reshape(B * S * M)
  return _run(idx_flat, table)
